# baseline (speedup 1.0000x reference)
import functools

import jax
import jax.numpy as jnp
from jax import lax
from jax.experimental import pallas as pl
from jax.experimental.pallas import tpu as pltpu
from jax.experimental.pallas import tpu_sc as plsc

N = 10000
D = 128
E = 320000

NW = 32
EPB = 128
CPT = 80
HCPT = CPT // 2
NCH = NW * CPT
EPAD = NCH * EPB
NROWS = 10112
RPT = NROWS // 16
NDEG = 10240
DPT = NDEG // 16

_mesh = plsc.VectorSubcoreMesh(core_axis_name="c", subcore_axis_name="s")


@functools.partial(
    pl.kernel,
    mesh=_mesh,
    out_type=jax.ShapeDtypeStruct((2 * NDEG,), jnp.float32),
    scratch_types=[
        pltpu.VMEM((CPT, EPB), jnp.int32),
        pltpu.VMEM((EPB,), jnp.float32),
        pltpu.VMEM((DPT,), jnp.float32),
        pltpu.VMEM_SHARED((NDEG,), jnp.float32),
    ],
)
def _sc_deg(dst_hbm, out_hbm, idx_v, ones_v, zb_v, deg_sh):
    c = lax.axis_index("c")
    s = lax.axis_index("s")
    wid = c * 16 + s

    for i in range(EPB // 16):
        ones_v[pl.ds(i * 16, 16)] = jnp.ones((16,), jnp.float32)
    for i in range(DPT // 16):
        zb_v[pl.ds(i * 16, 16)] = jnp.zeros((16,), jnp.float32)
    pltpu.sync_copy(zb_v, deg_sh.at[pl.ds(s * DPT, DPT)])
    plsc.subcore_barrier()
    pltpu.sync_copy(dst_hbm.at[pl.ds(wid * CPT, CPT)], idx_v)

    def body(j, carry):
        pltpu.sync_copy(ones_v, deg_sh.at[idx_v.at[j]], add=True)
        return carry

    lax.fori_loop(0, CPT, body, 0)
    plsc.subcore_barrier()
    pltpu.sync_copy(deg_sh.at[pl.ds(s * DPT, DPT)],
                    out_hbm.at[pl.ds(c * NDEG + s * DPT, DPT)])


def _newton_rsqrt(x):
    y = 1.0 / x
    for _ in range(12):
        y = y * (1.5 - 0.5 * x * y * y)
    return y


@functools.partial(
    pl.kernel,
    mesh=_mesh,
    out_type=[
        jax.ShapeDtypeStruct((2, NROWS, D), jnp.float32),
        jax.ShapeDtypeStruct((NROWS, D), jnp.float32),
    ],
    scratch_types=[
        pltpu.VMEM((HCPT, EPB), jnp.int32),
        pltpu.VMEM((HCPT, EPB), jnp.int32),
        pltpu.VMEM((EPB, D), jnp.float32),
        pltpu.VMEM((EPB, D), jnp.float32),
        pltpu.VMEM((DPT + 16,), jnp.float32),
        pltpu.VMEM((DPT,), jnp.float32),
        pltpu.VMEM_SHARED((NROWS, D), jnp.float32),
        pltpu.SemaphoreType.DMA,
        pltpu.SemaphoreType.DMA,
        pltpu.SemaphoreType.DMA,
        pltpu.SemaphoreType.DMA,
    ],
)
def _sc_spmm(h_hbm, deg_hbm, src_hbm, dst_hbm, out_hbm, g_hbm,
             si_v, di_v, rows_v, buf_b, dv0, dv1, acc_sh,
             sem, sem_b, sem_sa, sem_sb):
    c = lax.axis_index("c")
    s = lax.axis_index("s")
    wid = c * 16 + s

    def zrow(j, carry):
        for i in range(D // 16):
            rows_v[j, pl.ds(i * 16, 16)] = jnp.zeros((16,), jnp.float32)
        return carry

    lax.fori_loop(0, EPB, zrow, 0)
    for b in range(4):
        pltpu.sync_copy(rows_v, acc_sh.at[pl.ds(s * RPT + b * EPB, EPB)])
    pltpu.sync_copy(rows_v.at[pl.ds(0, RPT - 4 * EPB)],
                    acc_sh.at[pl.ds(s * RPT + 4 * EPB, RPT - 4 * EPB)])

    row0 = s * RPT
    pltpu.sync_copy(deg_hbm.at[pl.ds(row0, RPT)], dv0.at[pl.ds(0, RPT)])
    pltpu.sync_copy(deg_hbm.at[pl.ds(NDEG + row0, RPT)],
                    dv1.at[pl.ds(0, RPT)])
    for i in range(RPT // 16):
        x = dv0[pl.ds(i * 16, 16)] + dv1[pl.ds(i * 16, 16)] + 1.0
        dv0[pl.ds(i * 16, 16)] = _newton_rsqrt(x)
    for b in range((RPT + EPB - 1) // EPB):
        nr = min(EPB, RPT - b * EPB)
        pltpu.sync_copy(h_hbm.at[pl.ds(row0 + b * EPB, nr)],
                        buf_b.at[pl.ds(0, nr)])

        def grow(r, carry):
            dvec = dv0[pl.ds(b * EPB + r, 16)]
            dis = lax.broadcast_in_dim(dvec[0], (16,), ())
            for k in range(D // 16):
                buf_b[r, pl.ds(k * 16, 16)] = (
                    buf_b[r, pl.ds(k * 16, 16)] * dis)
            return carry

        lax.fori_loop(0, nr, grow, 0)
        pltpu.sync_copy(buf_b.at[pl.ds(0, nr)],
                        g_hbm.at[pl.ds(row0 + b * EPB, nr)])
    plsc.subcore_barrier()

    for half in range(2):
        pltpu.sync_copy(src_hbm.at[pl.ds(wid * CPT + half * HCPT, HCPT)],
                        si_v)
        pltpu.sync_copy(dst_hbm.at[pl.ds(wid * CPT + half * HCPT, HCPT)],
                        di_v)
        pltpu.async_copy(g_hbm.at[di_v.at[0]], rows_v, sem).wait()
        pltpu.async_copy(rows_v, acc_sh.at[si_v.at[0]], sem_sa, add=True)
        pltpu.async_copy(g_hbm.at[di_v.at[1]], buf_b, sem_b).wait()
        pltpu.async_copy(buf_b, acc_sh.at[si_v.at[1]], sem_sb, add=True)

        def body(i, carry):
            j = 2 * i + 2
            pltpu.make_async_copy(
                rows_v, acc_sh.at[si_v.at[j - 2]], sem_sa).wait()
            pltpu.async_copy(g_hbm.at[di_v.at[j]], rows_v, sem).wait()
            pltpu.async_copy(rows_v, acc_sh.at[si_v.at[j]], sem_sa,
                             add=True)
            pltpu.make_async_copy(
                buf_b, acc_sh.at[si_v.at[j - 1]], sem_sb).wait()
            pltpu.async_copy(g_hbm.at[di_v.at[j + 1]], buf_b, sem_b).wait()
            pltpu.async_copy(buf_b, acc_sh.at[si_v.at[j + 1]], sem_sb,
                             add=True)
            return carry

        lax.fori_loop(0, HCPT // 2 - 1, body, 0)
        pltpu.make_async_copy(
            rows_v, acc_sh.at[si_v.at[HCPT - 2]], sem_sa).wait()
        pltpu.make_async_copy(
            buf_b, acc_sh.at[si_v.at[HCPT - 1]], sem_sb).wait()
    plsc.subcore_barrier()
    pltpu.sync_copy(acc_sh.at[pl.ds(s * RPT, RPT)],
                    out_hbm.at[c, pl.ds(s * RPT, RPT)])


def _tc_final_body(h_ref, g_ref, acc_ref, d0_ref, d1_ref, c1_ref, c2_ref,
                   o_ref):
    dis = lax.rsqrt(d0_ref[...] + d1_ref[...] + 1.0)
    accsum = acc_ref[0] + acc_ref[1]
    a = dis * (accsum + g_ref[...])
    o_ref[...] = h_ref[...] * c1_ref[...] + a * c2_ref[...]


def _tc_final(h_pad, g_pad, acc, d0, d1, c1, c2):
    rb = N // 5
    return pl.pallas_call(
        _tc_final_body,
        grid=(5,),
        in_specs=[
            pl.BlockSpec((rb, D), lambda i: (i, 0)),
            pl.BlockSpec((rb, D), lambda i: (i, 0)),
            pl.BlockSpec((2, rb, D), lambda i: (0, i, 0)),
            pl.BlockSpec((rb, 1), lambda i: (i, 0)),
            pl.BlockSpec((rb, 1), lambda i: (i, 0)),
            pl.BlockSpec((1, D), lambda i: (0, 0)),
            pl.BlockSpec((1, D), lambda i: (0, 0)),
        ],
        out_specs=pl.BlockSpec((rb, D), lambda i: (i, 0)),
        out_shape=jax.ShapeDtypeStruct((N, D), jnp.float32),
    )(h_pad, g_pad, acc, d0, d1, c1, c2)


def kernel(h, edge_index, k_cross_channel, K_channel_wise):
    src = edge_index[0].astype(jnp.int32)
    dst = edge_index[1].astype(jnp.int32)
    pad = N + jnp.arange(EPAD - E, dtype=jnp.int32) % (NROWS - N)
    srcp = jnp.concatenate([src, pad]).reshape(NCH, EPB)
    dstp = jnp.concatenate([dst, pad]).reshape(NCH, EPB)
    h_pad = jnp.pad(h, ((0, NROWS - N), (0, 0)))

    deg_flat = _sc_deg(dstp)
    acc, g_pad = _sc_spmm(h_pad, deg_flat, srcp, dstp)

    d0 = deg_flat[:NROWS, None]
    d1 = deg_flat[NDEG:NDEG + NROWS, None]
    k = k_cross_channel[0]
    c1 = (2.0 - k - K_channel_wise) * 0.5
    c2 = (k + K_channel_wise) * 0.5
    return _tc_final(h_pad, g_pad, acc, d0, d1, c1, c2)


# speedup vs baseline: 1.0226x; 1.0226x over previous
import functools

import jax
import jax.numpy as jnp
from jax import lax
from jax.experimental import pallas as pl
from jax.experimental.pallas import tpu as pltpu
from jax.experimental.pallas import tpu_sc as plsc

N = 10000
D = 128
E = 320000

NW = 32
EPB = 128
CPT = 80
NCH = NW * CPT
EPAD = NCH * EPB
NROWS = 10112
RPT = NROWS // 16
NDEG = 10240
DPT = NDEG // 16

_mesh = plsc.VectorSubcoreMesh(core_axis_name="c", subcore_axis_name="s")


@functools.partial(
    pl.kernel,
    mesh=_mesh,
    out_type=jax.ShapeDtypeStruct((2 * NDEG,), jnp.float32),
    scratch_types=[
        pltpu.VMEM((NCH // NW, EPB), jnp.int32),
        pltpu.VMEM((EPB,), jnp.float32),
        pltpu.VMEM((DPT,), jnp.float32),
        pltpu.VMEM_SHARED((NDEG,), jnp.float32),
    ],
)
def _sc_deg(dst_hbm, out_hbm, idx_v, ones_v, zb_v, deg_sh):
    c = lax.axis_index("c")
    s = lax.axis_index("s")
    wid = c * 16 + s

    for i in range(EPB // 16):
        ones_v[pl.ds(i * 16, 16)] = jnp.ones((16,), jnp.float32)
    for i in range(DPT // 16):
        zb_v[pl.ds(i * 16, 16)] = jnp.zeros((16,), jnp.float32)
    pltpu.sync_copy(zb_v, deg_sh.at[pl.ds(s * DPT, DPT)])
    plsc.subcore_barrier()
    pltpu.sync_copy(dst_hbm.at[pl.ds(wid * CPT, CPT)], idx_v)

    def body(j, carry):
        pltpu.sync_copy(ones_v, deg_sh.at[idx_v.at[j]], add=True)
        return carry

    lax.fori_loop(0, CPT, body, 0)
    plsc.subcore_barrier()
    pltpu.sync_copy(deg_sh.at[pl.ds(s * DPT, DPT)],
                    out_hbm.at[pl.ds(c * NDEG + s * DPT, DPT)])


@functools.partial(
    pl.kernel,
    mesh=_mesh,
    out_type=jax.ShapeDtypeStruct((2, NROWS, D), jnp.float32),
    scratch_types=[
        pltpu.VMEM((CPT // 2, EPB), jnp.int32),
        pltpu.VMEM((CPT // 2, EPB), jnp.int32),
        pltpu.VMEM((EPB, D), jnp.float32),
        pltpu.VMEM((EPB, D), jnp.float32),
        pltpu.VMEM_SHARED((NROWS, D), jnp.float32),
        pltpu.SemaphoreType.DMA,
        pltpu.SemaphoreType.DMA,
        pltpu.SemaphoreType.DMA,
        pltpu.SemaphoreType.DMA,
    ],
)
def _sc_spmm(g_hbm, src_hbm, dst_hbm, out_hbm, si_v, di_v, rows_v, buf_b,
             acc_sh, sem, sem_b, sem_sa, sem_sb):
    c = lax.axis_index("c")
    s = lax.axis_index("s")
    wid = c * 16 + s

    def zrow(j, carry):
        for i in range(D // 16):
            rows_v[j, pl.ds(i * 16, 16)] = jnp.zeros((16,), jnp.float32)
        return carry

    lax.fori_loop(0, EPB, zrow, 0)
    for b in range(4):
        pltpu.sync_copy(rows_v, acc_sh.at[pl.ds(s * RPT + b * EPB, EPB)])
    pltpu.sync_copy(rows_v.at[pl.ds(0, RPT - 4 * EPB)],
                    acc_sh.at[pl.ds(s * RPT + 4 * EPB, RPT - 4 * EPB)])
    plsc.subcore_barrier()
    H = CPT // 2
    for half in range(2):
        pltpu.sync_copy(src_hbm.at[pl.ds(wid * CPT + half * H, H)], si_v)
        pltpu.sync_copy(dst_hbm.at[pl.ds(wid * CPT + half * H, H)], di_v)
        pltpu.async_copy(g_hbm.at[di_v.at[0]], rows_v, sem).wait()
        pltpu.async_copy(rows_v, acc_sh.at[si_v.at[0]], sem_sa, add=True)
        pltpu.async_copy(g_hbm.at[di_v.at[1]], buf_b, sem_b).wait()
        pltpu.async_copy(buf_b, acc_sh.at[si_v.at[1]], sem_sb, add=True)

        def body(i, carry):
            j = 2 * i + 2
            pltpu.make_async_copy(
                rows_v, acc_sh.at[si_v.at[j - 2]], sem_sa).wait()
            pltpu.async_copy(g_hbm.at[di_v.at[j]], rows_v, sem).wait()
            pltpu.async_copy(rows_v, acc_sh.at[si_v.at[j]], sem_sa, add=True)
            pltpu.make_async_copy(
                buf_b, acc_sh.at[si_v.at[j - 1]], sem_sb).wait()
            pltpu.async_copy(g_hbm.at[di_v.at[j + 1]], buf_b, sem_b).wait()
            pltpu.async_copy(
                buf_b, acc_sh.at[si_v.at[j + 1]], sem_sb, add=True)
            return carry

        lax.fori_loop(0, H // 2 - 1, body, 0)
        pltpu.make_async_copy(
            rows_v, acc_sh.at[si_v.at[H - 2]], sem_sa).wait()
        pltpu.make_async_copy(
            buf_b, acc_sh.at[si_v.at[H - 1]], sem_sb).wait()
    plsc.subcore_barrier()
    pltpu.sync_copy(acc_sh.at[pl.ds(s * RPT, RPT)],
                    out_hbm.at[c, pl.ds(s * RPT, RPT)])


_RB = 2528


def _tc_g_body(h_ref, d0_ref, d1_ref, g_ref, dis_ref):
    dis = lax.rsqrt(d0_ref[...] + d1_ref[...] + 1.0)
    g_ref[...] = h_ref[...] * dis
    dis_ref[...] = dis


def _tc_g(h_pad, d0, d1):
    grid = (NROWS // _RB,)
    return pl.pallas_call(
        _tc_g_body,
        grid=grid,
        in_specs=[
            pl.BlockSpec((_RB, D), lambda i: (i, 0)),
            pl.BlockSpec((_RB, 1), lambda i: (i, 0)),
            pl.BlockSpec((_RB, 1), lambda i: (i, 0)),
        ],
        out_specs=[
            pl.BlockSpec((_RB, D), lambda i: (i, 0)),
            pl.BlockSpec((_RB, 1), lambda i: (i, 0)),
        ],
        out_shape=[
            jax.ShapeDtypeStruct((NROWS, D), jnp.float32),
            jax.ShapeDtypeStruct((NROWS, 1), jnp.float32),
        ],
    )(h_pad, d0, d1)


def _tc_final_body(h_ref, g_ref, acc_ref, dis_ref, c1_ref, c2_ref, o_ref):
    accsum = acc_ref[0] + acc_ref[1]
    a = dis_ref[...] * (accsum + g_ref[...])
    o_ref[...] = h_ref[...] * c1_ref[...] + a * c2_ref[...]


def _tc_final(h_pad, g_pad, acc, dis_col, c1, c2):
    rb = N // 5
    return pl.pallas_call(
        _tc_final_body,
        grid=(5,),
        in_specs=[
            pl.BlockSpec((rb, D), lambda i: (i, 0)),
            pl.BlockSpec((rb, D), lambda i: (i, 0)),
            pl.BlockSpec((2, rb, D), lambda i: (0, i, 0)),
            pl.BlockSpec((rb, 1), lambda i: (i, 0)),
            pl.BlockSpec((1, D), lambda i: (0, 0)),
            pl.BlockSpec((1, D), lambda i: (0, 0)),
        ],
        out_specs=pl.BlockSpec((rb, D), lambda i: (i, 0)),
        out_shape=jax.ShapeDtypeStruct((N, D), jnp.float32),
    )(h_pad, g_pad, acc, dis_col, c1, c2)


def kernel(h, edge_index, k_cross_channel, K_channel_wise):
    src = edge_index[0].astype(jnp.int32)
    dst = edge_index[1].astype(jnp.int32)
    pad = N + jnp.arange(EPAD - E, dtype=jnp.int32) % (NROWS - N)
    srcp = jnp.concatenate([src, pad]).reshape(NCH, EPB)
    dstp = jnp.concatenate([dst, pad]).reshape(NCH, EPB)
    h_pad = jnp.pad(h, ((0, NROWS - N), (0, 0)))

    deg_flat = _sc_deg(dstp)
    d0 = deg_flat[:NROWS, None]
    d1 = deg_flat[NDEG:NDEG + NROWS, None]
    g_pad, dis_col = _tc_g(h_pad, d0, d1)
    acc = _sc_spmm(g_pad, srcp, dstp)

    k = k_cross_channel[0]
    c1 = (2.0 - k - K_channel_wise) * 0.5
    c2 = (k + K_channel_wise) * 0.5
    return _tc_final(h_pad, g_pad, acc, dis_col, c1, c2)
